# Initial kernel scaffold; baseline (speedup 1.0000x reference)
import jax
import jax.numpy as jnp
from jax.experimental import pallas as pl


def kernel(x, edge_index, edge_attr, batch, params):
    raise NotImplementedError("write your pallas kernel here")


# speedup vs baseline: 3.7005x; 3.7005x over previous
import functools

import jax
import jax.numpy as jnp
from jax import lax
from jax.experimental import pallas as pl
from jax.experimental.pallas import tpu as pltpu
from jax.experimental.pallas import tpu_sc as plsc

N_NODES_ = 10000
N_EDGES_ = 320000
NODE_DIM_ = 128
EDGE_DIM_ = 16
HID_ = 64
N_GRAPHS_ = 64
BN_EPS_ = 1e-5

NC_ = 2
NS_ = 16
NW_ = NC_ * NS_
EW_ = N_EDGES_ // NW_
EB_ = 80
NBLK_ = EW_ // EB_
NPAD_ = 10240
ROWS_PER_SUB_ = NPAD_ // NS_
ZROWS_ = 128


def _sc_agg(h, e, src3, dst3):
    mesh = plsc.VectorSubcoreMesh(core_axis_name="c", subcore_axis_name="s")

    @functools.partial(
        pl.kernel,
        out_type=jax.ShapeDtypeStruct((NC_, NPAD_, HID_), jnp.float32),
        mesh=mesh,
        scratch_types=[
            pltpu.VMEM((NBLK_, EB_), jnp.int32),
            pltpu.VMEM((NBLK_, EB_), jnp.int32),
            pltpu.VMEM((EB_, HID_), jnp.float32),
            pltpu.VMEM((EB_, HID_), jnp.float32),
            pltpu.VMEM((ZROWS_, HID_), jnp.float32),
            pltpu.VMEM_SHARED((NPAD_, HID_), jnp.float32),
            pltpu.SemaphoreType.DMA,
        ],
        compiler_params=pltpu.CompilerParams(use_tc_tiling_on_sc=False),
    )
    def k(h_hbm, e_hbm, src_hbm, dst_hbm, out_hbm,
          src_v, dst_v, hrow_v, msg_v, zb_v, agg_sh, sem):
        cid = lax.axis_index("c")
        sid = lax.axis_index("s")
        wid = sid * NC_ + cid

        pltpu.sync_copy(src_hbm.at[wid], src_v)
        pltpu.sync_copy(dst_hbm.at[wid], dst_v)

        @pl.loop(0, ZROWS_)
        def _(i):
            for c in range(HID_ // 16):
                zb_v[i, pl.ds(c * 16, 16)] = jnp.zeros((16,), jnp.float32)

        for t in range(ROWS_PER_SUB_ // ZROWS_):
            pltpu.sync_copy(
                zb_v, agg_sh.at[pl.ds(sid * ROWS_PER_SUB_ + t * ZROWS_, ZROWS_)])
        plsc.subcore_barrier()

        @pl.loop(0, NBLK_)
        def _(j):
            ebase = wid * EW_ + j * EB_
            pltpu.sync_copy(e_hbm.at[pl.ds(ebase, EB_)], msg_v)
            pltpu.async_copy(h_hbm.at[src_v.at[j]], hrow_v, sem).wait()

            @pl.loop(0, EB_)
            def _(i):
                for c in range(HID_ // 16):
                    sl = pl.ds(c * 16, 16)
                    msg_v[i, sl] = jnp.maximum(msg_v[i, sl] + hrow_v[i, sl], 0.0)

            pltpu.sync_copy(msg_v, agg_sh.at[dst_v.at[j]], add=True)

        plsc.subcore_barrier()
        for t in range(ROWS_PER_SUB_ // ZROWS_):
            r0 = sid * ROWS_PER_SUB_ + t * ZROWS_
            pltpu.sync_copy(agg_sh.at[pl.ds(r0, ZROWS_)], zb_v)
            pltpu.sync_copy(zb_v, out_hbm.at[cid, pl.ds(r0, ZROWS_)])

    return k(h, e, src3, dst3)


def _mm_bias(x, w, b):
    def body(x_ref, w_ref, b_ref, o_ref):
        o_ref[...] = jnp.dot(x_ref[...], w_ref[...],
                             preferred_element_type=jnp.float32) + b_ref[...]
    n, _ = x.shape
    h = w.shape[1]
    return pl.pallas_call(
        body,
        out_shape=jax.ShapeDtypeStruct((n, h), jnp.float32),
    )(x, w, b.reshape(1, h))


def _edge_embed(ea, w, b):
    nb = 32
    rows = N_EDGES_ // nb

    def body(ea_ref, w_ref, b_ref, o_ref):
        o_ref[...] = jnp.dot(ea_ref[...], w_ref[...],
                             preferred_element_type=jnp.float32) + b_ref[...]

    return pl.pallas_call(
        body,
        grid=(nb,),
        in_specs=[
            pl.BlockSpec((rows, EDGE_DIM_), lambda i: (i, 0)),
            pl.BlockSpec((EDGE_DIM_, HID_), lambda i: (0, 0)),
            pl.BlockSpec((1, HID_), lambda i: (0, 0)),
        ],
        out_specs=pl.BlockSpec((rows, HID_), lambda i: (i, 0)),
        out_shape=jax.ShapeDtypeStruct((N_EDGES_, HID_), jnp.float32),
    )(ea, w, b.reshape(1, HID_))


def _layer_update(h, agg, lyr):
    def body(h_ref, agg_ref, w1_ref, b1_ref, w2_ref, b2_ref, g_ref, bt_ref,
             o_ref):
        hh = h_ref[...]
        z = hh + agg_ref[0, :N_NODES_] + agg_ref[1, :N_NODES_]
        z = jnp.maximum(jnp.dot(z, w1_ref[...],
                                preferred_element_type=jnp.float32)
                        + b1_ref[...], 0.0)
        z = jnp.dot(z, w2_ref[...],
                    preferred_element_type=jnp.float32) + b2_ref[...]
        mean = jnp.mean(z, axis=0, keepdims=True)
        zc = z - mean
        var = jnp.mean(zc * zc, axis=0, keepdims=True)
        z = zc * lax.rsqrt(var + BN_EPS_) * g_ref[...] + bt_ref[...]
        o_ref[...] = jnp.maximum(z + hh, 0.0)

    return pl.pallas_call(
        body,
        out_shape=jax.ShapeDtypeStruct((N_NODES_, HID_), jnp.float32),
    )(h, agg, lyr["w1"], lyr["b1"].reshape(1, HID_),
      lyr["w2"], lyr["b2"].reshape(1, HID_),
      lyr["gamma"].reshape(1, HID_), lyr["beta"].reshape(1, HID_))


def _pool_head(h, batch_row, head):
    (w1, b1), (w2, b2), (w3, b3) = head

    def body(h_ref, b_ref, w1_ref, b1_ref, w2_ref, b2_ref, w3_ref, b3_ref,
             o_ref):
        gid = lax.broadcasted_iota(jnp.int32, (N_GRAPHS_, N_NODES_), 0)
        onehot_t = jnp.where(b_ref[...] == gid, 1.0, 0.0)
        hh = h_ref[...]
        add_pool = jnp.dot(onehot_t, hh, preferred_element_type=jnp.float32)
        cnt = jnp.sum(onehot_t, axis=1, keepdims=True)
        mean_pool = add_pool / jnp.maximum(cnt, 1.0)
        hp = jnp.concatenate([mean_pool, add_pool], axis=1)
        u = jnp.maximum(jnp.dot(hp, w1_ref[...],
                                preferred_element_type=jnp.float32)
                        + b1_ref[...], 0.0)
        u = jnp.maximum(jnp.dot(u, w2_ref[...],
                                preferred_element_type=jnp.float32)
                        + b2_ref[...], 0.0)
        o_ref[...] = jnp.dot(u, w3_ref[...],
                             preferred_element_type=jnp.float32) + b3_ref[...]

    nt = w3.shape[1]
    return pl.pallas_call(
        body,
        out_shape=jax.ShapeDtypeStruct((N_GRAPHS_, nt), jnp.float32),
    )(h, batch_row, w1, b1.reshape(1, -1), w2, b2.reshape(1, -1),
      w3, b3.reshape(1, -1))


def kernel(x, edge_index, edge_attr, batch, params):
    src3 = edge_index[0].astype(jnp.int32).reshape(NW_, NBLK_, EB_)
    dst3 = edge_index[1].astype(jnp.int32).reshape(NW_, NBLK_, EB_)
    batch_row = batch.astype(jnp.int32).reshape(1, N_NODES_)

    h = _mm_bias(x, params["node_w"], params["node_b"])
    e = _edge_embed(edge_attr, params["edge_w"], params["edge_b"])
    for lyr in params["layers"]:
        agg = _sc_agg(h, e, src3, dst3)
        h = _layer_update(h, agg, lyr)
    return _pool_head(h, batch_row, params["head"])


# speedup vs baseline: 6.6493x; 1.7969x over previous
import functools

import jax
import jax.numpy as jnp
from jax import lax
from jax.experimental import pallas as pl
from jax.experimental.pallas import tpu as pltpu
from jax.experimental.pallas import tpu_sc as plsc

N_NODES_ = 10000
N_EDGES_ = 320000
NODE_DIM_ = 128
EDGE_DIM_ = 16
HID_ = 64
N_GRAPHS_ = 64
BN_EPS_ = 1e-5

NC_ = 2
NS_ = 16
NW_ = NC_ * NS_
EW_ = N_EDGES_ // NW_
EB_ = 80
NBLK_ = EW_ // EB_
NPAD_ = 10240
ROWS_PER_SUB_ = NPAD_ // NS_
ZROWS_ = 128


def _sc_agg(h, e, src3, dst3):
    mesh = plsc.VectorSubcoreMesh(core_axis_name="c", subcore_axis_name="s")

    @functools.partial(
        pl.kernel,
        out_type=jax.ShapeDtypeStruct((NC_, NPAD_, HID_), jnp.float32),
        mesh=mesh,
        scratch_types=[
            pltpu.VMEM((NBLK_, EB_), jnp.int32),
            pltpu.VMEM((NBLK_, EB_), jnp.int32),
            pltpu.VMEM((EB_, HID_), jnp.float32),
            pltpu.VMEM((EB_, HID_), jnp.float32),
            pltpu.VMEM((EB_, HID_), jnp.float32),
            pltpu.VMEM((EB_, HID_), jnp.float32),
            pltpu.VMEM((EB_, HID_), jnp.float32),
            pltpu.VMEM((EB_, HID_), jnp.float32),
            pltpu.VMEM((ZROWS_, HID_), jnp.float32),
            pltpu.VMEM_SHARED((NPAD_, HID_), jnp.float32),
            pltpu.SemaphoreType.DMA,
            pltpu.SemaphoreType.DMA,
            pltpu.SemaphoreType.DMA,
            pltpu.SemaphoreType.DMA,
            pltpu.SemaphoreType.DMA,
            pltpu.SemaphoreType.DMA,
        ],
        compiler_params=pltpu.CompilerParams(use_tc_tiling_on_sc=False),
    )
    def k(h_hbm, e_hbm, src_hbm, dst_hbm, out_hbm,
          src_v, dst_v, eb0, eb1, hr0, hr1, mo0, mo1, zb_v, agg_sh,
          se0, se1, sg0, sg1, ss0, ss1):
        cid = lax.axis_index("c")
        sid = lax.axis_index("s")
        wid = sid * NC_ + cid

        ebuf = (eb0, eb1)
        hbuf = (hr0, hr1)
        mbuf = (mo0, mo1)
        sem_e = (se0, se1)
        sem_g = (sg0, sg1)
        sem_s = (ss0, ss1)

        def e_desc(t, b):
            return pltpu.make_async_copy(
                e_hbm.at[pl.ds(wid * EW_ + t * EB_, EB_)], ebuf[b], sem_e[b])

        def g_desc(t, b):
            return pltpu.make_async_copy(
                h_hbm.at[src_v.at[t]], hbuf[b], sem_g[b])

        def s_desc(t, b):
            return pltpu.make_async_copy(
                mbuf[b], agg_sh.at[dst_v.at[t]], sem_s[b])

        pltpu.sync_copy(src_hbm.at[wid], src_v)
        pltpu.sync_copy(dst_hbm.at[wid], dst_v)

        @pl.loop(0, ZROWS_)
        def _(i):
            for c in range(HID_ // 16):
                zb_v[i, pl.ds(c * 16, 16)] = jnp.zeros((16,), jnp.float32)

        for t in range(ROWS_PER_SUB_ // ZROWS_):
            pltpu.sync_copy(
                zb_v, agg_sh.at[pl.ds(sid * ROWS_PER_SUB_ + t * ZROWS_, ZROWS_)])
        plsc.subcore_barrier()

        e_desc(0, 0).start()
        g_desc(0, 0).start()

        @pl.loop(0, NBLK_, step=2)
        def _(j):
            for b in range(2):
                t = j + b

                @pl.when(t < NBLK_)
                def _():
                    @pl.when(t + 1 < NBLK_)
                    def _():
                        e_desc(t + 1, 1 - b).start()
                        g_desc(t + 1, 1 - b).start()

                    e_desc(t, b).wait()
                    g_desc(t, b).wait()

                    @pl.when(t >= 2)
                    def _():
                        s_desc(t - 2, b).wait()

                    eb = ebuf[b]
                    hb = hbuf[b]
                    mb = mbuf[b]

                    @pl.loop(0, EB_)
                    def _(i):
                        for c in range(HID_ // 16):
                            sl = pl.ds(c * 16, 16)
                            mb[i, sl] = jnp.maximum(eb[i, sl] + hb[i, sl], 0.0)

                    s_desc(t, b).start(add=True)

        s_desc(NBLK_ - 2, (NBLK_ - 2) % 2).wait()
        s_desc(NBLK_ - 1, (NBLK_ - 1) % 2).wait()

        plsc.subcore_barrier()
        for t in range(ROWS_PER_SUB_ // ZROWS_):
            r0 = sid * ROWS_PER_SUB_ + t * ZROWS_
            pltpu.sync_copy(agg_sh.at[pl.ds(r0, ZROWS_)], zb_v)
            pltpu.sync_copy(zb_v, out_hbm.at[cid, pl.ds(r0, ZROWS_)])

    return k(h, e, src3, dst3)


def _mm_bias(x, w, b):
    def body(x_ref, w_ref, b_ref, o_ref):
        o_ref[...] = jnp.dot(x_ref[...], w_ref[...],
                             preferred_element_type=jnp.float32) + b_ref[...]
    n, _ = x.shape
    h = w.shape[1]
    return pl.pallas_call(
        body,
        out_shape=jax.ShapeDtypeStruct((n, h), jnp.float32),
    )(x, w, b.reshape(1, h))


def _edge_embed(ea, w, b):
    nb = 32
    rows = N_EDGES_ // nb

    def body(ea_ref, w_ref, b_ref, o_ref):
        o_ref[...] = jnp.dot(ea_ref[...], w_ref[...],
                             preferred_element_type=jnp.float32) + b_ref[...]

    return pl.pallas_call(
        body,
        grid=(nb,),
        in_specs=[
            pl.BlockSpec((rows, EDGE_DIM_), lambda i: (i, 0)),
            pl.BlockSpec((EDGE_DIM_, HID_), lambda i: (0, 0)),
            pl.BlockSpec((1, HID_), lambda i: (0, 0)),
        ],
        out_specs=pl.BlockSpec((rows, HID_), lambda i: (i, 0)),
        out_shape=jax.ShapeDtypeStruct((N_EDGES_, HID_), jnp.float32),
    )(ea, w, b.reshape(1, HID_))


def _layer_update(h, agg, lyr):
    def body(h_ref, agg_ref, w1_ref, b1_ref, w2_ref, b2_ref, g_ref, bt_ref,
             o_ref):
        hh = h_ref[...]
        z = hh + agg_ref[0, :N_NODES_] + agg_ref[1, :N_NODES_]
        z = jnp.maximum(jnp.dot(z, w1_ref[...],
                                preferred_element_type=jnp.float32)
                        + b1_ref[...], 0.0)
        z = jnp.dot(z, w2_ref[...],
                    preferred_element_type=jnp.float32) + b2_ref[...]
        mean = jnp.mean(z, axis=0, keepdims=True)
        zc = z - mean
        var = jnp.mean(zc * zc, axis=0, keepdims=True)
        z = zc * lax.rsqrt(var + BN_EPS_) * g_ref[...] + bt_ref[...]
        o_ref[...] = jnp.maximum(z + hh, 0.0)

    return pl.pallas_call(
        body,
        out_shape=jax.ShapeDtypeStruct((N_NODES_, HID_), jnp.float32),
    )(h, agg, lyr["w1"], lyr["b1"].reshape(1, HID_),
      lyr["w2"], lyr["b2"].reshape(1, HID_),
      lyr["gamma"].reshape(1, HID_), lyr["beta"].reshape(1, HID_))


def _pool_head(h, batch_row, head):
    (w1, b1), (w2, b2), (w3, b3) = head

    def body(h_ref, b_ref, w1_ref, b1_ref, w2_ref, b2_ref, w3_ref, b3_ref,
             o_ref):
        gid = lax.broadcasted_iota(jnp.int32, (N_GRAPHS_, N_NODES_), 0)
        onehot_t = jnp.where(b_ref[...] == gid, 1.0, 0.0)
        hh = h_ref[...]
        add_pool = jnp.dot(onehot_t, hh, preferred_element_type=jnp.float32)
        cnt = jnp.sum(onehot_t, axis=1, keepdims=True)
        mean_pool = add_pool / jnp.maximum(cnt, 1.0)
        hp = jnp.concatenate([mean_pool, add_pool], axis=1)
        u = jnp.maximum(jnp.dot(hp, w1_ref[...],
                                preferred_element_type=jnp.float32)
                        + b1_ref[...], 0.0)
        u = jnp.maximum(jnp.dot(u, w2_ref[...],
                                preferred_element_type=jnp.float32)
                        + b2_ref[...], 0.0)
        o_ref[...] = jnp.dot(u, w3_ref[...],
                             preferred_element_type=jnp.float32) + b3_ref[...]

    nt = w3.shape[1]
    return pl.pallas_call(
        body,
        out_shape=jax.ShapeDtypeStruct((N_GRAPHS_, nt), jnp.float32),
    )(h, batch_row, w1, b1.reshape(1, -1), w2, b2.reshape(1, -1),
      w3, b3.reshape(1, -1))


def kernel(x, edge_index, edge_attr, batch, params):
    src3 = edge_index[0].astype(jnp.int32).reshape(NW_, NBLK_, EB_)
    dst3 = edge_index[1].astype(jnp.int32).reshape(NW_, NBLK_, EB_)
    batch_row = batch.astype(jnp.int32).reshape(1, N_NODES_)

    h = _mm_bias(x, params["node_w"], params["node_b"])
    e = _edge_embed(edge_attr, params["edge_w"], params["edge_b"])
    for lyr in params["layers"]:
        agg = _sc_agg(h, e, src3, dst3)
        h = _layer_update(h, agg, lyr)
    return _pool_head(h, batch_row, params["head"])


# speedup vs baseline: 6.7389x; 1.0135x over previous
import functools

import jax
import jax.numpy as jnp
from jax import lax
from jax.experimental import pallas as pl
from jax.experimental.pallas import tpu as pltpu
from jax.experimental.pallas import tpu_sc as plsc

N_NODES_ = 10000
N_EDGES_ = 320000
NODE_DIM_ = 128
EDGE_DIM_ = 16
HID_ = 64
N_GRAPHS_ = 64
BN_EPS_ = 1e-5

NC_ = 2
NS_ = 16
NW_ = NC_ * NS_
EW_ = N_EDGES_ // NW_
EB_ = 80
NBLK_ = EW_ // EB_
NPAD_ = 10240
ROWS_PER_SUB_ = NPAD_ // NS_
ZROWS_ = 128


def _sc_agg(h, e, src3, dst3):
    mesh = plsc.VectorSubcoreMesh(core_axis_name="c", subcore_axis_name="s")

    @functools.partial(
        pl.kernel,
        out_type=jax.ShapeDtypeStruct((NC_, NPAD_, HID_), jnp.float32),
        mesh=mesh,
        scratch_types=[
            pltpu.VMEM((NBLK_, EB_), jnp.int32),
            pltpu.VMEM((NBLK_, EB_), jnp.int32),
            pltpu.VMEM((EB_, HID_), jnp.float32),
            pltpu.VMEM((EB_, HID_), jnp.float32),
            pltpu.VMEM((EB_, HID_), jnp.float32),
            pltpu.VMEM((EB_, HID_), jnp.float32),
            pltpu.VMEM((EB_, HID_), jnp.float32),
            pltpu.VMEM((EB_, HID_), jnp.float32),
            pltpu.VMEM((ZROWS_, HID_), jnp.float32),
            pltpu.VMEM_SHARED((NPAD_, HID_), jnp.float32),
            pltpu.SemaphoreType.DMA,
            pltpu.SemaphoreType.DMA,
            pltpu.SemaphoreType.DMA,
            pltpu.SemaphoreType.DMA,
            pltpu.SemaphoreType.DMA,
            pltpu.SemaphoreType.DMA,
        ],
        compiler_params=pltpu.CompilerParams(use_tc_tiling_on_sc=False),
    )
    def k(h_hbm, e_hbm, src_hbm, dst_hbm, out_hbm,
          src_v, dst_v, eb0, eb1, hr0, hr1, mo0, mo1, zb_v, agg_sh,
          se0, se1, sg0, sg1, ss0, ss1):
        cid = lax.axis_index("c")
        sid = lax.axis_index("s")
        wid = sid * NC_ + cid

        ebuf = (eb0, eb1)
        hbuf = (hr0, hr1)
        mbuf = (mo0, mo1)
        sem_e = (se0, se1)
        sem_g = (sg0, sg1)
        sem_s = (ss0, ss1)

        def e_desc(t, b):
            return pltpu.make_async_copy(
                e_hbm.at[pl.ds(wid * EW_ + t * EB_, EB_)], ebuf[b], sem_e[b])

        def g_desc(t, b):
            return pltpu.make_async_copy(
                h_hbm.at[src_v.at[t]], hbuf[b], sem_g[b])

        def s_desc(t, b):
            return pltpu.make_async_copy(
                mbuf[b], agg_sh.at[dst_v.at[t]], sem_s[b])

        pltpu.sync_copy(src_hbm.at[wid], src_v)
        pltpu.sync_copy(dst_hbm.at[wid], dst_v)

        @pl.loop(0, ZROWS_)
        def _(i):
            for c in range(HID_ // 16):
                zb_v[i, pl.ds(c * 16, 16)] = jnp.zeros((16,), jnp.float32)

        for t in range(ROWS_PER_SUB_ // ZROWS_):
            pltpu.sync_copy(
                zb_v, agg_sh.at[pl.ds(sid * ROWS_PER_SUB_ + t * ZROWS_, ZROWS_)])
        plsc.subcore_barrier()

        e_desc(0, 0).start()
        g_desc(0, 0).start()

        @pl.loop(0, NBLK_, step=2)
        def _(j):
            for b in range(2):
                t = j + b

                @pl.when(t < NBLK_)
                def _():
                    @pl.when(t + 1 < NBLK_)
                    def _():
                        e_desc(t + 1, 1 - b).start()
                        g_desc(t + 1, 1 - b).start()

                    e_desc(t, b).wait()
                    g_desc(t, b).wait()

                    @pl.when(t >= 2)
                    def _():
                        s_desc(t - 2, b).wait()

                    eb = ebuf[b]
                    hb = hbuf[b]
                    mb = mbuf[b]

                    @pl.loop(0, EB_)
                    def _(i):
                        for c in range(HID_ // 16):
                            sl = pl.ds(c * 16, 16)
                            mb[i, sl] = jnp.maximum(eb[i, sl] + hb[i, sl], 0.0)

                    s_desc(t, b).start(add=True)

        s_desc(NBLK_ - 2, (NBLK_ - 2) % 2).wait()
        s_desc(NBLK_ - 1, (NBLK_ - 1) % 2).wait()

        plsc.subcore_barrier()
        for t in range(ROWS_PER_SUB_ // ZROWS_):
            r0 = sid * ROWS_PER_SUB_ + t * ZROWS_
            pltpu.sync_copy(agg_sh.at[pl.ds(r0, ZROWS_)], zb_v)
            pltpu.sync_copy(zb_v, out_hbm.at[cid, pl.ds(r0, ZROWS_)])

    return k(h, e, src3, dst3)


def _embed(x, nw, nb_, ea, ew, eb_):
    g = 25
    nrows = N_NODES_ // g
    erows = N_EDGES_ // g

    def body(x_ref, nw_ref, nb_ref, ea_ref, ew_ref, eb_ref, h_ref, e_ref):
        h_ref[...] = jnp.dot(x_ref[...], nw_ref[...],
                             preferred_element_type=jnp.float32) + nb_ref[...]
        e_ref[...] = jnp.dot(ea_ref[...], ew_ref[...],
                             preferred_element_type=jnp.float32) + eb_ref[...]

    return pl.pallas_call(
        body,
        grid=(g,),
        in_specs=[
            pl.BlockSpec((nrows, NODE_DIM_), lambda i: (i, 0)),
            pl.BlockSpec((NODE_DIM_, HID_), lambda i: (0, 0)),
            pl.BlockSpec((1, HID_), lambda i: (0, 0)),
            pl.BlockSpec((erows, EDGE_DIM_), lambda i: (i, 0)),
            pl.BlockSpec((EDGE_DIM_, HID_), lambda i: (0, 0)),
            pl.BlockSpec((1, HID_), lambda i: (0, 0)),
        ],
        out_specs=[
            pl.BlockSpec((nrows, HID_), lambda i: (i, 0)),
            pl.BlockSpec((erows, HID_), lambda i: (i, 0)),
        ],
        out_shape=[
            jax.ShapeDtypeStruct((N_NODES_, HID_), jnp.float32),
            jax.ShapeDtypeStruct((N_EDGES_, HID_), jnp.float32),
        ],
    )(x, nw, nb_.reshape(1, HID_), ea, ew, eb_.reshape(1, HID_))


def _layer_update(h, agg, lyr, head=None, batch_row=None):
    def _core(h_ref, agg_ref, w1_ref, b1_ref, w2_ref, b2_ref, g_ref, bt_ref):
        hh = h_ref[...]
        z = hh + agg_ref[0, :N_NODES_] + agg_ref[1, :N_NODES_]
        z = jnp.maximum(jnp.dot(z, w1_ref[...],
                                preferred_element_type=jnp.float32)
                        + b1_ref[...], 0.0)
        z = jnp.dot(z, w2_ref[...],
                    preferred_element_type=jnp.float32) + b2_ref[...]
        mean = jnp.mean(z, axis=0, keepdims=True)
        zc = z - mean
        var = jnp.mean(zc * zc, axis=0, keepdims=True)
        z = zc * lax.rsqrt(var + BN_EPS_) * g_ref[...] + bt_ref[...]
        return jnp.maximum(z + hh, 0.0)

    lyr_args = (h, agg, lyr["w1"], lyr["b1"].reshape(1, HID_),
                lyr["w2"], lyr["b2"].reshape(1, HID_),
                lyr["gamma"].reshape(1, HID_), lyr["beta"].reshape(1, HID_))

    if head is None:
        def body(h_ref, agg_ref, w1_ref, b1_ref, w2_ref, b2_ref, g_ref,
                 bt_ref, o_ref):
            o_ref[...] = _core(h_ref, agg_ref, w1_ref, b1_ref, w2_ref, b2_ref,
                               g_ref, bt_ref)

        return pl.pallas_call(
            body,
            out_shape=jax.ShapeDtypeStruct((N_NODES_, HID_), jnp.float32),
        )(*lyr_args)

    (hw1, hb1), (hw2, hb2), (hw3, hb3) = head
    nt = hw3.shape[1]

    def body(h_ref, agg_ref, w1_ref, b1_ref, w2_ref, b2_ref, g_ref, bt_ref,
             batch_ref, hw1_ref, hb1_ref, hw2_ref, hb2_ref, hw3_ref, hb3_ref,
             o_ref):
        hn = _core(h_ref, agg_ref, w1_ref, b1_ref, w2_ref, b2_ref, g_ref,
                   bt_ref)
        gid = lax.broadcasted_iota(jnp.int32, (N_GRAPHS_, N_NODES_), 0)
        onehot_t = jnp.where(batch_ref[...] == gid, 1.0, 0.0)
        add_pool = jnp.dot(onehot_t, hn, preferred_element_type=jnp.float32)
        cnt = jnp.sum(onehot_t, axis=1, keepdims=True)
        mean_pool = add_pool / jnp.maximum(cnt, 1.0)
        hp = jnp.concatenate([mean_pool, add_pool], axis=1)
        u = jnp.maximum(jnp.dot(hp, hw1_ref[...],
                                preferred_element_type=jnp.float32)
                        + hb1_ref[...], 0.0)
        u = jnp.maximum(jnp.dot(u, hw2_ref[...],
                                preferred_element_type=jnp.float32)
                        + hb2_ref[...], 0.0)
        o_ref[...] = jnp.dot(u, hw3_ref[...],
                             preferred_element_type=jnp.float32) + hb3_ref[...]

    return pl.pallas_call(
        body,
        out_shape=jax.ShapeDtypeStruct((N_GRAPHS_, nt), jnp.float32),
    )(*lyr_args, batch_row, hw1, hb1.reshape(1, -1), hw2, hb2.reshape(1, -1),
      hw3, hb3.reshape(1, -1))


def kernel(x, edge_index, edge_attr, batch, params):
    src3 = edge_index[0].astype(jnp.int32).reshape(NW_, NBLK_, EB_)
    dst3 = edge_index[1].astype(jnp.int32).reshape(NW_, NBLK_, EB_)
    batch_row = batch.astype(jnp.int32).reshape(1, N_NODES_)

    h, e = _embed(x, params["node_w"], params["node_b"],
                  edge_attr, params["edge_w"], params["edge_b"])
    n_layers = len(params["layers"])
    for li, lyr in enumerate(params["layers"]):
        agg = _sc_agg(h, e, src3, dst3)
        if li + 1 < n_layers:
            h = _layer_update(h, agg, lyr)
        else:
            return _layer_update(h, agg, lyr,
                                 head=params["head"], batch_row=batch_row)


# speedup vs baseline: 8.7900x; 1.3044x over previous
import functools

import numpy as np

import jax
import jax.numpy as jnp
from jax import lax
from jax.experimental import pallas as pl
from jax.experimental.pallas import tpu as pltpu
from jax.experimental.pallas import tpu_sc as plsc

N_NODES_ = 10000
N_EDGES_ = 320000
NODE_DIM_ = 128
EDGE_DIM_ = 16
HID_ = 64
N_GRAPHS_ = 64
BN_EPS_ = 1e-5

NC_ = 2
NS_ = 16
NW_ = NC_ * NS_
EW_ = N_EDGES_ // NW_
EB_ = 80
NBLK_ = EW_ // EB_
NPAD_ = 10240
ROWS_PER_SUB_ = NPAD_ // NS_
ZROWS_ = 128

EG_ = 25
E4_ = N_EDGES_ // (4 * EG_)
HPK_ = HID_ // 2


def _edge_perm():
    f = np.arange(N_EDGES_, dtype=np.int64)
    i = f // (4 * E4_)
    rem = f % (4 * E4_)
    return np.asarray((4 * i + rem % 4) * E4_ + rem // 4, dtype=np.int32)


def _rtne_hi(x):
    bits = lax.bitcast_convert_type(x, jnp.uint32)
    return (bits + jnp.uint32(0x7FFF) + ((bits >> jnp.uint32(16))
                                         & jnp.uint32(1))) >> jnp.uint32(16)


def _pack2(lo, hi):
    return _rtne_hi(lo) | (_rtne_hi(hi) << jnp.uint32(16))


def _sc_agg(hbf, e4, src3, dst3):
    mesh = plsc.VectorSubcoreMesh(core_axis_name="c", subcore_axis_name="s")

    @functools.partial(
        pl.kernel,
        out_type=jax.ShapeDtypeStruct((NC_, NPAD_, HID_), jnp.float32),
        mesh=mesh,
        scratch_types=[
            pltpu.VMEM((NBLK_, EB_), jnp.int32),
            pltpu.VMEM((NBLK_, EB_), jnp.int32),
            pltpu.VMEM((EB_ // 4, 128), jnp.uint32),
            pltpu.VMEM((EB_ // 4, 128), jnp.uint32),
            pltpu.VMEM((EB_, HPK_), jnp.uint32),
            pltpu.VMEM((EB_, HPK_), jnp.uint32),
            pltpu.VMEM((EB_, HID_), jnp.float32),
            pltpu.VMEM((EB_, HID_), jnp.float32),
            pltpu.VMEM((ZROWS_, HID_), jnp.float32),
            pltpu.VMEM_SHARED((NPAD_, HID_), jnp.float32),
            pltpu.SemaphoreType.DMA,
            pltpu.SemaphoreType.DMA,
            pltpu.SemaphoreType.DMA,
            pltpu.SemaphoreType.DMA,
            pltpu.SemaphoreType.DMA,
            pltpu.SemaphoreType.DMA,
        ],
        compiler_params=pltpu.CompilerParams(use_tc_tiling_on_sc=False,
                                             needs_layout_passes=False),
    )
    def k(h_hbm, e_hbm, src_hbm, dst_hbm, out_hbm,
          src_v, dst_v, eb0, eb1, hr0, hr1, mo0, mo1, zb_v, agg_sh,
          se0, se1, sg0, sg1, ss0, ss1):
        cid = lax.axis_index("c")
        sid = lax.axis_index("s")
        wid = sid * NC_ + cid

        ebuf = (eb0, eb1)
        hbuf = (hr0, hr1)
        mbuf = (mo0, mo1)
        sem_e = (se0, se1)
        sem_g = (sg0, sg1)
        sem_s = (ss0, ss1)

        sh16 = jnp.full((16,), 16, jnp.uint32)
        mhi = jnp.full((16,), 0xFFFF0000, jnp.uint32)

        def lo_f(x):
            return plsc.bitcast(x << sh16, jnp.float32)

        def hi_f(x):
            return plsc.bitcast(x & mhi, jnp.float32)

        def e_desc(t, b):
            return pltpu.make_async_copy(
                e_hbm.at[pl.ds(wid * (EW_ // 4) + t * (EB_ // 4), EB_ // 4)],
                ebuf[b], sem_e[b])

        def g_desc(t, b):
            return pltpu.make_async_copy(
                h_hbm.at[src_v.at[t]], hbuf[b], sem_g[b])

        def s_desc(t, b):
            return pltpu.make_async_copy(
                mbuf[b], agg_sh.at[dst_v.at[t]], sem_s[b])

        pltpu.sync_copy(src_hbm.at[wid], src_v)
        pltpu.sync_copy(dst_hbm.at[wid], dst_v)

        @pl.loop(0, ZROWS_)
        def _(i):
            for c in range(HID_ // 16):
                zb_v[i, pl.ds(c * 16, 16)] = jnp.zeros((16,), jnp.float32)

        for t in range(ROWS_PER_SUB_ // ZROWS_):
            pltpu.sync_copy(
                zb_v, agg_sh.at[pl.ds(sid * ROWS_PER_SUB_ + t * ZROWS_, ZROWS_)])
        plsc.subcore_barrier()

        e_desc(0, 0).start()
        g_desc(0, 0).start()

        @pl.loop(0, NBLK_, step=2)
        def _(j):
            for b in range(2):
                t = j + b

                @pl.when(t < NBLK_)
                def _():
                    @pl.when(t + 1 < NBLK_)
                    def _():
                        e_desc(t + 1, 1 - b).start()
                        g_desc(t + 1, 1 - b).start()

                    e_desc(t, b).wait()
                    g_desc(t, b).wait()

                    @pl.when(t >= 2)
                    def _():
                        s_desc(t - 2, b).wait()

                    eb = ebuf[b]
                    hb = hbuf[b]
                    mb = mbuf[b]

                    @pl.loop(0, EB_ // 4)
                    def _(r):
                        for half in range(2):
                            xes = [eb[r, pl.ds(half * 64 + c * 16, 16)]
                                   for c in range(4)]
                            for part in range(2):
                                m = 4 * r + 2 * half + part
                                hw0 = hb[m, pl.ds(0, 16)]
                                hw1 = hb[m, pl.ds(16, 16)]
                                hch = (lo_f(hw0), lo_f(hw1),
                                       hi_f(hw0), hi_f(hw1))
                                for c in range(4):
                                    ef = lo_f(xes[c]) if part == 0 \
                                        else hi_f(xes[c])
                                    mb[m, pl.ds(c * 16, 16)] = jnp.maximum(
                                        hch[c] + ef, 0.0)

                    s_desc(t, b).start(add=True)

        s_desc(NBLK_ - 2, (NBLK_ - 2) % 2).wait()
        s_desc(NBLK_ - 1, (NBLK_ - 1) % 2).wait()

        plsc.subcore_barrier()
        for t in range(ROWS_PER_SUB_ // ZROWS_):
            r0 = sid * ROWS_PER_SUB_ + t * ZROWS_
            pltpu.sync_copy(agg_sh.at[pl.ds(r0, ZROWS_)], zb_v)
            pltpu.sync_copy(zb_v, out_hbm.at[cid, pl.ds(r0, ZROWS_)])

    return k(hbf, e4, src3, dst3)


def _embed(x, nw, nb_, ea, ew, eb_):
    nrows = N_NODES_ // EG_

    def body(x_ref, nw_ref, nb_ref, ea0_ref, ea1_ref, ea2_ref, ea3_ref,
             ew_ref, eb_ref, h_ref, hbf_ref, e_ref):
        h = jnp.dot(x_ref[...], nw_ref[...],
                    preferred_element_type=jnp.float32) + nb_ref[...]
        h_ref[...] = h
        rh = _rtne_hi(h)
        hbf_ref[...] = rh[:, :HPK_] | (rh[:, HPK_:] << jnp.uint32(16))

        dn = (((0,), (0,)), ((), ()))
        es = [lax.dot_general(r[...], ew_ref[...], dn,
                              preferred_element_type=jnp.float32) + eb_ref[...]
              for r in (ea0_ref, ea1_ref, ea2_ref, ea3_ref)]
        e_ref[...] = jnp.concatenate(
            [_pack2(es[0], es[1]), _pack2(es[2], es[3])], axis=1)

    ea_t = ea.T

    return pl.pallas_call(
        body,
        grid=(EG_,),
        in_specs=[
            pl.BlockSpec((nrows, NODE_DIM_), lambda i: (i, 0)),
            pl.BlockSpec((NODE_DIM_, HID_), lambda i: (0, 0)),
            pl.BlockSpec((1, HID_), lambda i: (0, 0)),
            pl.BlockSpec((EDGE_DIM_, E4_), lambda i: (0, 4 * i)),
            pl.BlockSpec((EDGE_DIM_, E4_), lambda i: (0, 4 * i + 1)),
            pl.BlockSpec((EDGE_DIM_, E4_), lambda i: (0, 4 * i + 2)),
            pl.BlockSpec((EDGE_DIM_, E4_), lambda i: (0, 4 * i + 3)),
            pl.BlockSpec((EDGE_DIM_, HID_), lambda i: (0, 0)),
            pl.BlockSpec((1, HID_), lambda i: (0, 0)),
        ],
        out_specs=[
            pl.BlockSpec((nrows, HID_), lambda i: (i, 0)),
            pl.BlockSpec((nrows, HPK_), lambda i: (i, 0)),
            pl.BlockSpec((E4_, 128), lambda i: (i, 0)),
        ],
        out_shape=[
            jax.ShapeDtypeStruct((N_NODES_, HID_), jnp.float32),
            jax.ShapeDtypeStruct((N_NODES_, HPK_), jnp.uint32),
            jax.ShapeDtypeStruct((N_EDGES_ // 4, 128), jnp.uint32),
        ],
    )(x, nw, nb_.reshape(1, HID_), ea_t, ea_t, ea_t, ea_t,
      ew, eb_.reshape(1, HID_))


def _layer_update(h, agg, lyr, head=None, batch_row=None):
    def _core(h_ref, agg_ref, w1_ref, b1_ref, w2_ref, b2_ref, g_ref, bt_ref):
        hh = h_ref[...]
        z = hh + agg_ref[0, :N_NODES_] + agg_ref[1, :N_NODES_]
        z = jnp.maximum(jnp.dot(z, w1_ref[...],
                                preferred_element_type=jnp.float32)
                        + b1_ref[...], 0.0)
        z = jnp.dot(z, w2_ref[...],
                    preferred_element_type=jnp.float32) + b2_ref[...]
        mean = jnp.mean(z, axis=0, keepdims=True)
        zc = z - mean
        var = jnp.mean(zc * zc, axis=0, keepdims=True)
        z = zc * lax.rsqrt(var + BN_EPS_) * g_ref[...] + bt_ref[...]
        return jnp.maximum(z + hh, 0.0)

    lyr_args = (h, agg, lyr["w1"], lyr["b1"].reshape(1, HID_),
                lyr["w2"], lyr["b2"].reshape(1, HID_),
                lyr["gamma"].reshape(1, HID_), lyr["beta"].reshape(1, HID_))

    if head is None:
        def body(h_ref, agg_ref, w1_ref, b1_ref, w2_ref, b2_ref, g_ref,
                 bt_ref, o_ref, obf_ref):
            hn = _core(h_ref, agg_ref, w1_ref, b1_ref, w2_ref, b2_ref,
                       g_ref, bt_ref)
            o_ref[...] = hn
            rh = _rtne_hi(hn)
            obf_ref[...] = rh[:, :HPK_] | (rh[:, HPK_:] << jnp.uint32(16))

        return pl.pallas_call(
            body,
            out_shape=[
                jax.ShapeDtypeStruct((N_NODES_, HID_), jnp.float32),
                jax.ShapeDtypeStruct((N_NODES_, HPK_), jnp.uint32),
            ],
        )(*lyr_args)

    (hw1, hb1), (hw2, hb2), (hw3, hb3) = head
    nt = hw3.shape[1]

    def body(h_ref, agg_ref, w1_ref, b1_ref, w2_ref, b2_ref, g_ref, bt_ref,
             batch_ref, hw1_ref, hb1_ref, hw2_ref, hb2_ref, hw3_ref, hb3_ref,
             o_ref):
        hn = _core(h_ref, agg_ref, w1_ref, b1_ref, w2_ref, b2_ref, g_ref,
                   bt_ref)
        gid = lax.broadcasted_iota(jnp.int32, (N_GRAPHS_, N_NODES_), 0)
        onehot_t = jnp.where(batch_ref[...] == gid, 1.0, 0.0)
        add_pool = jnp.dot(onehot_t, hn, preferred_element_type=jnp.float32)
        cnt = jnp.sum(onehot_t, axis=1, keepdims=True)
        mean_pool = add_pool / jnp.maximum(cnt, 1.0)
        hp = jnp.concatenate([mean_pool, add_pool], axis=1)
        u = jnp.maximum(jnp.dot(hp, hw1_ref[...],
                                preferred_element_type=jnp.float32)
                        + hb1_ref[...], 0.0)
        u = jnp.maximum(jnp.dot(u, hw2_ref[...],
                                preferred_element_type=jnp.float32)
                        + hb2_ref[...], 0.0)
        o_ref[...] = jnp.dot(u, hw3_ref[...],
                             preferred_element_type=jnp.float32) + hb3_ref[...]

    return pl.pallas_call(
        body,
        out_shape=jax.ShapeDtypeStruct((N_GRAPHS_, nt), jnp.float32),
    )(*lyr_args, batch_row, hw1, hb1.reshape(1, -1), hw2, hb2.reshape(1, -1),
      hw3, hb3.reshape(1, -1))


def kernel(x, edge_index, edge_attr, batch, params):
    eidx = jnp.asarray(_edge_perm())
    src3 = edge_index[0].astype(jnp.int32)[eidx].reshape(NW_, NBLK_, EB_)
    dst3 = edge_index[1].astype(jnp.int32)[eidx].reshape(NW_, NBLK_, EB_)
    batch_row = batch.astype(jnp.int32).reshape(1, N_NODES_)

    h, hbf, e4 = _embed(x, params["node_w"], params["node_b"],
                        edge_attr, params["edge_w"], params["edge_b"])
    n_layers = len(params["layers"])
    for li, lyr in enumerate(params["layers"]):
        agg = _sc_agg(hbf, e4, src3, dst3)
        if li + 1 < n_layers:
            h, hbf = _layer_update(h, agg, lyr)
        else:
            return _layer_update(h, agg, lyr,
                                 head=params["head"], batch_row=batch_row)


# speedup vs baseline: 8.9608x; 1.0194x over previous
import functools

import numpy as np

import jax
import jax.numpy as jnp
from jax import lax
from jax.experimental import pallas as pl
from jax.experimental.pallas import tpu as pltpu
from jax.experimental.pallas import tpu_sc as plsc

N_NODES_ = 10000
N_EDGES_ = 320000
NODE_DIM_ = 128
EDGE_DIM_ = 16
HID_ = 64
N_GRAPHS_ = 64
BN_EPS_ = 1e-5

NC_ = 2
NS_ = 16
NW_ = NC_ * NS_
EW_ = N_EDGES_ // NW_
EB_ = 80
NBLK_ = EW_ // EB_
NPAD_ = 10240
ROWS_PER_SUB_ = NPAD_ // NS_
ZROWS_ = 128

EG_ = 25
E4_ = N_EDGES_ // (4 * EG_)
HPK_ = HID_ // 2


def _edge_perm():
    f = np.arange(N_EDGES_, dtype=np.int64)
    i = f // (4 * E4_)
    rem = f % (4 * E4_)
    return np.asarray((4 * i + rem % 4) * E4_ + rem // 4, dtype=np.int32)


def _rtne_hi(x):
    bits = lax.bitcast_convert_type(x, jnp.uint32)
    return (bits + jnp.uint32(0x7FFF) + ((bits >> jnp.uint32(16))
                                         & jnp.uint32(1))) >> jnp.uint32(16)


def _pack2(lo, hi):
    return _rtne_hi(lo) | (_rtne_hi(hi) << jnp.uint32(16))


def _sc_agg(hbf, e4, src3, dst3):
    mesh = plsc.VectorSubcoreMesh(core_axis_name="c", subcore_axis_name="s")

    @functools.partial(
        pl.kernel,
        out_type=jax.ShapeDtypeStruct((NC_, NPAD_, HID_), jnp.float32),
        mesh=mesh,
        scratch_types=[
            pltpu.VMEM((NBLK_, EB_), jnp.int32),
            pltpu.VMEM((NBLK_, EB_), jnp.int32),
            pltpu.VMEM((EB_ // 4, 128), jnp.uint32),
            pltpu.VMEM((EB_ // 4, 128), jnp.uint32),
            pltpu.VMEM((EB_ // 4, 128), jnp.uint32),
            pltpu.VMEM((EB_ // 4, 128), jnp.uint32),
            pltpu.VMEM((EB_, HPK_), jnp.uint32),
            pltpu.VMEM((EB_, HPK_), jnp.uint32),
            pltpu.VMEM((EB_, HPK_), jnp.uint32),
            pltpu.VMEM((EB_, HPK_), jnp.uint32),
            pltpu.VMEM((EB_, HID_), jnp.float32),
            pltpu.VMEM((EB_, HID_), jnp.float32),
            pltpu.VMEM((EB_, HID_), jnp.float32),
            pltpu.VMEM((EB_, HID_), jnp.float32),
            pltpu.VMEM((ZROWS_, HID_), jnp.float32),
            pltpu.VMEM_SHARED((NPAD_, HID_), jnp.float32),
            pltpu.SemaphoreType.DMA,
            pltpu.SemaphoreType.DMA,
            pltpu.SemaphoreType.DMA,
            pltpu.SemaphoreType.DMA,
            pltpu.SemaphoreType.DMA,
            pltpu.SemaphoreType.DMA,
            pltpu.SemaphoreType.DMA,
            pltpu.SemaphoreType.DMA,
            pltpu.SemaphoreType.DMA,
            pltpu.SemaphoreType.DMA,
            pltpu.SemaphoreType.DMA,
            pltpu.SemaphoreType.DMA,
        ],
        compiler_params=pltpu.CompilerParams(use_tc_tiling_on_sc=False,
                                             needs_layout_passes=False),
    )
    def k(h_hbm, e_hbm, src_hbm, dst_hbm, out_hbm,
          src_v, dst_v, eb0, eb1, eb2, eb3, hr0, hr1, hr2, hr3,
          mo0, mo1, mo2, mo3, zb_v, agg_sh,
          se0, se1, se2, se3, sg0, sg1, sg2, sg3, ss0, ss1, ss2, ss3):
        cid = lax.axis_index("c")
        sid = lax.axis_index("s")
        wid = sid * NC_ + cid

        ebuf = (eb0, eb1, eb2, eb3)
        hbuf = (hr0, hr1, hr2, hr3)
        mbuf = (mo0, mo1, mo2, mo3)
        sem_e = (se0, se1, se2, se3)
        sem_g = (sg0, sg1, sg2, sg3)
        sem_s = (ss0, ss1, ss2, ss3)

        sh16 = jnp.full((16,), 16, jnp.uint32)
        mhi = jnp.full((16,), 0xFFFF0000, jnp.uint32)

        def lo_f(x):
            return plsc.bitcast(x << sh16, jnp.float32)

        def hi_f(x):
            return plsc.bitcast(x & mhi, jnp.float32)

        def e_desc(t, b):
            return pltpu.make_async_copy(
                e_hbm.at[pl.ds(wid * (EW_ // 4) + t * (EB_ // 4), EB_ // 4)],
                ebuf[b], sem_e[b])

        def g_desc(t, b):
            return pltpu.make_async_copy(
                h_hbm.at[src_v.at[t]], hbuf[b], sem_g[b])

        def s_desc(t, b):
            return pltpu.make_async_copy(
                mbuf[b], agg_sh.at[dst_v.at[t]], sem_s[b])

        pltpu.sync_copy(src_hbm.at[wid], src_v)
        pltpu.sync_copy(dst_hbm.at[wid], dst_v)

        @pl.loop(0, ZROWS_)
        def _(i):
            for c in range(HID_ // 16):
                zb_v[i, pl.ds(c * 16, 16)] = jnp.zeros((16,), jnp.float32)

        for t in range(ROWS_PER_SUB_ // ZROWS_):
            pltpu.sync_copy(
                zb_v, agg_sh.at[pl.ds(sid * ROWS_PER_SUB_ + t * ZROWS_, ZROWS_)])
        plsc.subcore_barrier()

        for p in range(3):
            e_desc(p, p).start()
            g_desc(p, p).start()

        @pl.loop(0, NBLK_, step=4)
        def _(j):
            for b in range(4):
                t = j + b

                @pl.when(t < NBLK_)
                def _():
                    @pl.when(t + 3 < NBLK_)
                    def _():
                        e_desc(t + 3, (b + 3) % 4).start()
                        g_desc(t + 3, (b + 3) % 4).start()

                    e_desc(t, b).wait()
                    g_desc(t, b).wait()

                    @pl.when(t >= 4)
                    def _():
                        s_desc(t - 4, b).wait()

                    eb = ebuf[b]
                    hb = hbuf[b]
                    mb = mbuf[b]

                    @pl.loop(0, EB_ // 4)
                    def _(r):
                        for half in range(2):
                            xes = [eb[r, pl.ds(half * 64 + c * 16, 16)]
                                   for c in range(4)]
                            for part in range(2):
                                m = 4 * r + 2 * half + part
                                hw0 = hb[m, pl.ds(0, 16)]
                                hw1 = hb[m, pl.ds(16, 16)]
                                hch = (lo_f(hw0), lo_f(hw1),
                                       hi_f(hw0), hi_f(hw1))
                                for c in range(4):
                                    ef = lo_f(xes[c]) if part == 0 \
                                        else hi_f(xes[c])
                                    mb[m, pl.ds(c * 16, 16)] = jnp.maximum(
                                        hch[c] + ef, 0.0)

                    s_desc(t, b).start(add=True)

        for t in range(NBLK_ - 4, NBLK_):
            s_desc(t, t % 4).wait()

        plsc.subcore_barrier()
        for t in range(ROWS_PER_SUB_ // ZROWS_):
            r0 = sid * ROWS_PER_SUB_ + t * ZROWS_
            pltpu.sync_copy(agg_sh.at[pl.ds(r0, ZROWS_)], zb_v)
            pltpu.sync_copy(zb_v, out_hbm.at[cid, pl.ds(r0, ZROWS_)])

    return k(hbf, e4, src3, dst3)


def _embed(x, nw, nb_, ea, ew, eb_):
    nrows = N_NODES_ // EG_

    def body(x_ref, nw_ref, nb_ref, ea0_ref, ea1_ref, ea2_ref, ea3_ref,
             ew_ref, eb_ref, h_ref, hbf_ref, e_ref):
        h = jnp.dot(x_ref[...], nw_ref[...],
                    preferred_element_type=jnp.float32) + nb_ref[...]
        h_ref[...] = h
        rh = _rtne_hi(h)
        hbf_ref[...] = rh[:, :HPK_] | (rh[:, HPK_:] << jnp.uint32(16))

        dn = (((0,), (0,)), ((), ()))
        es = [lax.dot_general(r[...], ew_ref[...], dn,
                              preferred_element_type=jnp.float32) + eb_ref[...]
              for r in (ea0_ref, ea1_ref, ea2_ref, ea3_ref)]
        e_ref[...] = jnp.concatenate(
            [_pack2(es[0], es[1]), _pack2(es[2], es[3])], axis=1)

    ea_t = ea.T

    return pl.pallas_call(
        body,
        grid=(EG_,),
        in_specs=[
            pl.BlockSpec((nrows, NODE_DIM_), lambda i: (i, 0)),
            pl.BlockSpec((NODE_DIM_, HID_), lambda i: (0, 0)),
            pl.BlockSpec((1, HID_), lambda i: (0, 0)),
            pl.BlockSpec((EDGE_DIM_, E4_), lambda i: (0, 4 * i)),
            pl.BlockSpec((EDGE_DIM_, E4_), lambda i: (0, 4 * i + 1)),
            pl.BlockSpec((EDGE_DIM_, E4_), lambda i: (0, 4 * i + 2)),
            pl.BlockSpec((EDGE_DIM_, E4_), lambda i: (0, 4 * i + 3)),
            pl.BlockSpec((EDGE_DIM_, HID_), lambda i: (0, 0)),
            pl.BlockSpec((1, HID_), lambda i: (0, 0)),
        ],
        out_specs=[
            pl.BlockSpec((nrows, HID_), lambda i: (i, 0)),
            pl.BlockSpec((nrows, HPK_), lambda i: (i, 0)),
            pl.BlockSpec((E4_, 128), lambda i: (i, 0)),
        ],
        out_shape=[
            jax.ShapeDtypeStruct((N_NODES_, HID_), jnp.float32),
            jax.ShapeDtypeStruct((N_NODES_, HPK_), jnp.uint32),
            jax.ShapeDtypeStruct((N_EDGES_ // 4, 128), jnp.uint32),
        ],
    )(x, nw, nb_.reshape(1, HID_), ea_t, ea_t, ea_t, ea_t,
      ew, eb_.reshape(1, HID_))


def _layer_update(h, agg, lyr, head=None, batch_row=None):
    def _core(h_ref, agg_ref, w1_ref, b1_ref, w2_ref, b2_ref, g_ref, bt_ref):
        hh = h_ref[...]
        z = hh + agg_ref[0, :N_NODES_] + agg_ref[1, :N_NODES_]
        z = jnp.maximum(jnp.dot(z, w1_ref[...],
                                preferred_element_type=jnp.float32)
                        + b1_ref[...], 0.0)
        z = jnp.dot(z, w2_ref[...],
                    preferred_element_type=jnp.float32) + b2_ref[...]
        mean = jnp.mean(z, axis=0, keepdims=True)
        zc = z - mean
        var = jnp.mean(zc * zc, axis=0, keepdims=True)
        z = zc * lax.rsqrt(var + BN_EPS_) * g_ref[...] + bt_ref[...]
        return jnp.maximum(z + hh, 0.0)

    lyr_args = (h, agg, lyr["w1"], lyr["b1"].reshape(1, HID_),
                lyr["w2"], lyr["b2"].reshape(1, HID_),
                lyr["gamma"].reshape(1, HID_), lyr["beta"].reshape(1, HID_))

    if head is None:
        def body(h_ref, agg_ref, w1_ref, b1_ref, w2_ref, b2_ref, g_ref,
                 bt_ref, o_ref, obf_ref):
            hn = _core(h_ref, agg_ref, w1_ref, b1_ref, w2_ref, b2_ref,
                       g_ref, bt_ref)
            o_ref[...] = hn
            rh = _rtne_hi(hn)
            obf_ref[...] = rh[:, :HPK_] | (rh[:, HPK_:] << jnp.uint32(16))

        return pl.pallas_call(
            body,
            out_shape=[
                jax.ShapeDtypeStruct((N_NODES_, HID_), jnp.float32),
                jax.ShapeDtypeStruct((N_NODES_, HPK_), jnp.uint32),
            ],
        )(*lyr_args)

    (hw1, hb1), (hw2, hb2), (hw3, hb3) = head
    nt = hw3.shape[1]

    def body(h_ref, agg_ref, w1_ref, b1_ref, w2_ref, b2_ref, g_ref, bt_ref,
             batch_ref, hw1_ref, hb1_ref, hw2_ref, hb2_ref, hw3_ref, hb3_ref,
             o_ref):
        hn = _core(h_ref, agg_ref, w1_ref, b1_ref, w2_ref, b2_ref, g_ref,
                   bt_ref)
        gid = lax.broadcasted_iota(jnp.int32, (N_GRAPHS_, N_NODES_), 0)
        onehot_t = jnp.where(batch_ref[...] == gid, 1.0, 0.0)
        add_pool = jnp.dot(onehot_t, hn, preferred_element_type=jnp.float32)
        cnt = jnp.sum(onehot_t, axis=1, keepdims=True)
        mean_pool = add_pool / jnp.maximum(cnt, 1.0)
        hp = jnp.concatenate([mean_pool, add_pool], axis=1)
        u = jnp.maximum(jnp.dot(hp, hw1_ref[...],
                                preferred_element_type=jnp.float32)
                        + hb1_ref[...], 0.0)
        u = jnp.maximum(jnp.dot(u, hw2_ref[...],
                                preferred_element_type=jnp.float32)
                        + hb2_ref[...], 0.0)
        o_ref[...] = jnp.dot(u, hw3_ref[...],
                             preferred_element_type=jnp.float32) + hb3_ref[...]

    return pl.pallas_call(
        body,
        out_shape=jax.ShapeDtypeStruct((N_GRAPHS_, nt), jnp.float32),
    )(*lyr_args, batch_row, hw1, hb1.reshape(1, -1), hw2, hb2.reshape(1, -1),
      hw3, hb3.reshape(1, -1))


def kernel(x, edge_index, edge_attr, batch, params):
    eidx = jnp.asarray(_edge_perm())
    src3 = edge_index[0].astype(jnp.int32)[eidx].reshape(NW_, NBLK_, EB_)
    dst3 = edge_index[1].astype(jnp.int32)[eidx].reshape(NW_, NBLK_, EB_)
    batch_row = batch.astype(jnp.int32).reshape(1, N_NODES_)

    h, hbf, e4 = _embed(x, params["node_w"], params["node_b"],
                        edge_attr, params["edge_w"], params["edge_b"])
    n_layers = len(params["layers"])
    for li, lyr in enumerate(params["layers"]):
        agg = _sc_agg(hbf, e4, src3, dst3)
        if li + 1 < n_layers:
            h, hbf = _layer_update(h, agg, lyr)
        else:
            return _layer_update(h, agg, lyr,
                                 head=params["head"], batch_row=batch_row)


# speedup vs baseline: 9.0021x; 1.0046x over previous
import functools

import numpy as np

import jax
import jax.numpy as jnp
from jax import lax
from jax.experimental import pallas as pl
from jax.experimental.pallas import tpu as pltpu
from jax.experimental.pallas import tpu_sc as plsc

N_NODES_ = 10000
N_EDGES_ = 320000
NODE_DIM_ = 128
EDGE_DIM_ = 16
HID_ = 64
N_GRAPHS_ = 64
BN_EPS_ = 1e-5

NC_ = 2
NS_ = 16
NW_ = NC_ * NS_
EW_ = N_EDGES_ // NW_
EB_ = 80
NBLK_ = EW_ // EB_
NPAD_ = 10240
ROWS_PER_SUB_ = NPAD_ // NS_
ZROWS_ = 128

EG_ = 25
E4_ = N_EDGES_ // (4 * EG_)
HPK_ = HID_ // 2


def _edge_perm():
    f = np.arange(N_EDGES_, dtype=np.int64)
    i = f // (4 * E4_)
    rem = f % (4 * E4_)
    return np.asarray((4 * i + rem % 4) * E4_ + rem // 4, dtype=np.int32)


def _rtne_hi(x):
    bits = lax.bitcast_convert_type(x, jnp.uint32)
    return (bits + jnp.uint32(0x7FFF) + ((bits >> jnp.uint32(16))
                                         & jnp.uint32(1))) >> jnp.uint32(16)


def _pack2(lo, hi):
    return _rtne_hi(lo) | (_rtne_hi(hi) << jnp.uint32(16))


def _dot16(a, b, dn=None):
    a16 = a.astype(jnp.bfloat16)
    b16 = b.astype(jnp.bfloat16)
    if dn is None:
        return jnp.dot(a16, b16, preferred_element_type=jnp.float32)
    return lax.dot_general(a16, b16, dn, preferred_element_type=jnp.float32)


def _sc_agg(hbf, e4, src3, dst3):
    mesh = plsc.VectorSubcoreMesh(core_axis_name="c", subcore_axis_name="s")

    @functools.partial(
        pl.kernel,
        out_type=jax.ShapeDtypeStruct((NC_, NPAD_, HID_), jnp.float32),
        mesh=mesh,
        scratch_types=[
            pltpu.VMEM((NBLK_, EB_), jnp.int32),
            pltpu.VMEM((NBLK_, EB_), jnp.int32),
            pltpu.VMEM((EB_ // 4, 128), jnp.uint32),
            pltpu.VMEM((EB_ // 4, 128), jnp.uint32),
            pltpu.VMEM((EB_ // 4, 128), jnp.uint32),
            pltpu.VMEM((EB_ // 4, 128), jnp.uint32),
            pltpu.VMEM((EB_, HPK_), jnp.uint32),
            pltpu.VMEM((EB_, HPK_), jnp.uint32),
            pltpu.VMEM((EB_, HPK_), jnp.uint32),
            pltpu.VMEM((EB_, HPK_), jnp.uint32),
            pltpu.VMEM((EB_, HID_), jnp.float32),
            pltpu.VMEM((EB_, HID_), jnp.float32),
            pltpu.VMEM((EB_, HID_), jnp.float32),
            pltpu.VMEM((EB_, HID_), jnp.float32),
            pltpu.VMEM((ZROWS_, HID_), jnp.float32),
            pltpu.VMEM_SHARED((NPAD_, HID_), jnp.float32),
            pltpu.SemaphoreType.DMA,
            pltpu.SemaphoreType.DMA,
            pltpu.SemaphoreType.DMA,
            pltpu.SemaphoreType.DMA,
            pltpu.SemaphoreType.DMA,
            pltpu.SemaphoreType.DMA,
            pltpu.SemaphoreType.DMA,
            pltpu.SemaphoreType.DMA,
            pltpu.SemaphoreType.DMA,
            pltpu.SemaphoreType.DMA,
            pltpu.SemaphoreType.DMA,
            pltpu.SemaphoreType.DMA,
        ],
        compiler_params=pltpu.CompilerParams(use_tc_tiling_on_sc=False,
                                             needs_layout_passes=False),
    )
    def k(h_hbm, e_hbm, src_hbm, dst_hbm, out_hbm,
          src_v, dst_v, eb0, eb1, eb2, eb3, hr0, hr1, hr2, hr3,
          mo0, mo1, mo2, mo3, zb_v, agg_sh,
          se0, se1, se2, se3, sg0, sg1, sg2, sg3, ss0, ss1, ss2, ss3):
        cid = lax.axis_index("c")
        sid = lax.axis_index("s")
        wid = sid * NC_ + cid

        ebuf = (eb0, eb1, eb2, eb3)
        hbuf = (hr0, hr1, hr2, hr3)
        mbuf = (mo0, mo1, mo2, mo3)
        sem_e = (se0, se1, se2, se3)
        sem_g = (sg0, sg1, sg2, sg3)
        sem_s = (ss0, ss1, ss2, ss3)

        sh16 = jnp.full((16,), 16, jnp.uint32)
        mhi = jnp.full((16,), 0xFFFF0000, jnp.uint32)

        def lo_f(x):
            return plsc.bitcast(x << sh16, jnp.float32)

        def hi_f(x):
            return plsc.bitcast(x & mhi, jnp.float32)

        def e_desc(t, b):
            return pltpu.make_async_copy(
                e_hbm.at[pl.ds(wid * (EW_ // 4) + t * (EB_ // 4), EB_ // 4)],
                ebuf[b], sem_e[b])

        def g_desc(t, b):
            return pltpu.make_async_copy(
                h_hbm.at[src_v.at[t]], hbuf[b], sem_g[b])

        def s_desc(t, b):
            return pltpu.make_async_copy(
                mbuf[b], agg_sh.at[dst_v.at[t]], sem_s[b])

        pltpu.sync_copy(src_hbm.at[wid], src_v)
        pltpu.sync_copy(dst_hbm.at[wid], dst_v)

        @pl.loop(0, ZROWS_)
        def _(i):
            for c in range(HID_ // 16):
                zb_v[i, pl.ds(c * 16, 16)] = jnp.zeros((16,), jnp.float32)

        for t in range(ROWS_PER_SUB_ // ZROWS_):
            pltpu.sync_copy(
                zb_v, agg_sh.at[pl.ds(sid * ROWS_PER_SUB_ + t * ZROWS_, ZROWS_)])
        plsc.subcore_barrier()

        for p in range(3):
            e_desc(p, p).start()
            g_desc(p, p).start()

        @pl.loop(0, NBLK_, step=4)
        def _(j):
            for b in range(4):
                t = j + b

                @pl.when(t < NBLK_)
                def _():
                    @pl.when(t + 3 < NBLK_)
                    def _():
                        e_desc(t + 3, (b + 3) % 4).start()
                        g_desc(t + 3, (b + 3) % 4).start()

                    e_desc(t, b).wait()
                    g_desc(t, b).wait()

                    @pl.when(t >= 4)
                    def _():
                        s_desc(t - 4, b).wait()

                    eb = ebuf[b]
                    hb = hbuf[b]
                    mb = mbuf[b]

                    @pl.loop(0, EB_ // 4)
                    def _(r):
                        for half in range(2):
                            xes = [eb[r, pl.ds(half * 64 + c * 16, 16)]
                                   for c in range(4)]
                            for part in range(2):
                                m = 4 * r + 2 * half + part
                                hw0 = hb[m, pl.ds(0, 16)]
                                hw1 = hb[m, pl.ds(16, 16)]
                                hch = (lo_f(hw0), lo_f(hw1),
                                       hi_f(hw0), hi_f(hw1))
                                for c in range(4):
                                    ef = lo_f(xes[c]) if part == 0 \
                                        else hi_f(xes[c])
                                    mb[m, pl.ds(c * 16, 16)] = jnp.maximum(
                                        hch[c] + ef, 0.0)

                    s_desc(t, b).start(add=True)

        for t in range(NBLK_ - 4, NBLK_):
            s_desc(t, t % 4).wait()

        plsc.subcore_barrier()
        for t in range(ROWS_PER_SUB_ // ZROWS_):
            r0 = sid * ROWS_PER_SUB_ + t * ZROWS_
            pltpu.sync_copy(agg_sh.at[pl.ds(r0, ZROWS_)], zb_v)
            pltpu.sync_copy(zb_v, out_hbm.at[cid, pl.ds(r0, ZROWS_)])

    return k(hbf, e4, src3, dst3)


def _embed(x, nw, nb_, ea, ew, eb_):
    nrows = N_NODES_ // EG_

    def body(x_ref, nw_ref, nb_ref, ea0_ref, ea1_ref, ea2_ref, ea3_ref,
             ew_ref, eb_ref, h_ref, hbf_ref, e_ref):
        h = _dot16(x_ref[...], nw_ref[...]) + nb_ref[...]
        h_ref[...] = h
        rh = _rtne_hi(h)
        hbf_ref[...] = rh[:, :HPK_] | (rh[:, HPK_:] << jnp.uint32(16))

        dn = (((0,), (0,)), ((), ()))
        es = [_dot16(r[...], ew_ref[...], dn) + eb_ref[...]
              for r in (ea0_ref, ea1_ref, ea2_ref, ea3_ref)]
        e_ref[...] = jnp.concatenate(
            [_pack2(es[0], es[1]), _pack2(es[2], es[3])], axis=1)

    ea_t = ea.T

    return pl.pallas_call(
        body,
        grid=(EG_,),
        in_specs=[
            pl.BlockSpec((nrows, NODE_DIM_), lambda i: (i, 0)),
            pl.BlockSpec((NODE_DIM_, HID_), lambda i: (0, 0)),
            pl.BlockSpec((1, HID_), lambda i: (0, 0)),
            pl.BlockSpec((EDGE_DIM_, E4_), lambda i: (0, 4 * i)),
            pl.BlockSpec((EDGE_DIM_, E4_), lambda i: (0, 4 * i + 1)),
            pl.BlockSpec((EDGE_DIM_, E4_), lambda i: (0, 4 * i + 2)),
            pl.BlockSpec((EDGE_DIM_, E4_), lambda i: (0, 4 * i + 3)),
            pl.BlockSpec((EDGE_DIM_, HID_), lambda i: (0, 0)),
            pl.BlockSpec((1, HID_), lambda i: (0, 0)),
        ],
        out_specs=[
            pl.BlockSpec((nrows, HID_), lambda i: (i, 0)),
            pl.BlockSpec((nrows, HPK_), lambda i: (i, 0)),
            pl.BlockSpec((E4_, 128), lambda i: (i, 0)),
        ],
        out_shape=[
            jax.ShapeDtypeStruct((N_NODES_, HID_), jnp.float32),
            jax.ShapeDtypeStruct((N_NODES_, HPK_), jnp.uint32),
            jax.ShapeDtypeStruct((N_EDGES_ // 4, 128), jnp.uint32),
        ],
    )(x, nw, nb_.reshape(1, HID_), ea_t, ea_t, ea_t, ea_t,
      ew, eb_.reshape(1, HID_))


def _layer_update(h, agg, lyr, head=None, batch_row=None):
    def _core(h_ref, agg_ref, w1_ref, b1_ref, w2_ref, b2_ref, g_ref, bt_ref):
        hh = h_ref[...]
        z = hh + agg_ref[0, :N_NODES_] + agg_ref[1, :N_NODES_]
        z = jnp.maximum(_dot16(z, w1_ref[...]) + b1_ref[...], 0.0)
        z = _dot16(z, w2_ref[...]) + b2_ref[...]
        mean = jnp.mean(z, axis=0, keepdims=True)
        zc = z - mean
        var = jnp.mean(zc * zc, axis=0, keepdims=True)
        z = zc / jnp.sqrt(var + BN_EPS_) * g_ref[...] + bt_ref[...]
        return jnp.maximum(z + hh, 0.0)

    lyr_args = (h, agg, lyr["w1"], lyr["b1"].reshape(1, HID_),
                lyr["w2"], lyr["b2"].reshape(1, HID_),
                lyr["gamma"].reshape(1, HID_), lyr["beta"].reshape(1, HID_))

    if head is None:
        def body(h_ref, agg_ref, w1_ref, b1_ref, w2_ref, b2_ref, g_ref,
                 bt_ref, o_ref, obf_ref):
            hn = _core(h_ref, agg_ref, w1_ref, b1_ref, w2_ref, b2_ref,
                       g_ref, bt_ref)
            o_ref[...] = hn
            rh = _rtne_hi(hn)
            obf_ref[...] = rh[:, :HPK_] | (rh[:, HPK_:] << jnp.uint32(16))

        return pl.pallas_call(
            body,
            out_shape=[
                jax.ShapeDtypeStruct((N_NODES_, HID_), jnp.float32),
                jax.ShapeDtypeStruct((N_NODES_, HPK_), jnp.uint32),
            ],
        )(*lyr_args)

    (hw1, hb1), (hw2, hb2), (hw3, hb3) = head
    nt = hw3.shape[1]

    def body(h_ref, agg_ref, w1_ref, b1_ref, w2_ref, b2_ref, g_ref, bt_ref,
             batch_ref, hw1_ref, hb1_ref, hw2_ref, hb2_ref, hw3_ref, hb3_ref,
             o_ref):
        hn = _core(h_ref, agg_ref, w1_ref, b1_ref, w2_ref, b2_ref, g_ref,
                   bt_ref)
        gid = lax.broadcasted_iota(jnp.int32, (N_GRAPHS_, N_NODES_), 0)
        onehot_t = jnp.where(batch_ref[...] == gid, 1.0, 0.0)
        add_pool = jnp.dot(onehot_t, hn, preferred_element_type=jnp.float32)
        cnt = jnp.sum(onehot_t, axis=1, keepdims=True)
        mean_pool = add_pool / jnp.maximum(cnt, 1.0)
        hp = jnp.concatenate([mean_pool, add_pool], axis=1)
        u = jnp.maximum(_dot16(hp, hw1_ref[...]) + hb1_ref[...], 0.0)
        u = jnp.maximum(_dot16(u, hw2_ref[...]) + hb2_ref[...], 0.0)
        o_ref[...] = _dot16(u, hw3_ref[...]) + hb3_ref[...]

    return pl.pallas_call(
        body,
        out_shape=jax.ShapeDtypeStruct((N_GRAPHS_, nt), jnp.float32),
    )(*lyr_args, batch_row, hw1, hb1.reshape(1, -1), hw2, hb2.reshape(1, -1),
      hw3, hb3.reshape(1, -1))


def kernel(x, edge_index, edge_attr, batch, params):
    eidx = jnp.asarray(_edge_perm())
    src3 = edge_index[0].astype(jnp.int32)[eidx].reshape(NW_, NBLK_, EB_)
    dst3 = edge_index[1].astype(jnp.int32)[eidx].reshape(NW_, NBLK_, EB_)
    batch_row = batch.astype(jnp.int32).reshape(1, N_NODES_)

    h, hbf, e4 = _embed(x, params["node_w"], params["node_b"],
                        edge_attr, params["edge_w"], params["edge_b"])
    n_layers = len(params["layers"])
    for li, lyr in enumerate(params["layers"]):
        agg = _sc_agg(hbf, e4, src3, dst3)
        if li + 1 < n_layers:
            h, hbf = _layer_update(h, agg, lyr)
        else:
            return _layer_update(h, agg, lyr,
                                 head=params["head"], batch_row=batch_row)
